# baseline (speedup 1.0000x reference)
import math

import jax
import jax.numpy as jnp
from jax.experimental import pallas as pl
from jax.experimental.pallas import tpu as pltpu


def _embedder_kernel(t_ref, freqs_ref, w1_ref, b1_ref, w2_ref, b2_ref, o_ref):
    t_col = t_ref[...]
    args = t_col * freqs_ref[...]
    emb = jnp.concatenate(
        [jnp.cos(args), jnp.sin(args)], axis=1
    ).astype(jnp.bfloat16)

    h = (jnp.dot(emb, w1_ref[...], preferred_element_type=jnp.float32)
         + b1_ref[...])
    h = h * jax.lax.logistic(h)
    hb = h.astype(jnp.bfloat16)

    o_ref[...] = (jnp.dot(hb, w2_ref[...], preferred_element_type=jnp.float32)
                  + b2_ref[...])


def kernel(t, w1, b1, w2, b2, *, frequency_embedding_size=256,
           max_period=10000, max_tile_n=512):
    N = t.shape[0]
    F = frequency_embedding_size
    half = F // 2
    H = w1.shape[1]
    assert F % 2 == 0, "frequency_embedding_size must be even"
    assert w1.shape[0] == F and w2.shape == (H, H)

    freqs = jnp.exp(
        -math.log(max_period) * jnp.arange(half, dtype=jnp.float32) / half
    ).reshape(1, half)

    w1_bf = w1.astype(jnp.bfloat16)
    w2_bf = w2.astype(jnp.bfloat16)

    tn = min(max_tile_n, -(-N // 8) * 8)
    n_pad = -(-N // tn) * tn
    t_col = jnp.zeros((n_pad, 1), jnp.float32).at[:N, 0].set(
        t.astype(jnp.float32))

    out = pl.pallas_call(
        _embedder_kernel,
        grid=(n_pad // tn,),
        in_specs=[
            pl.BlockSpec((tn, 1), lambda i: (i, 0)),
            pl.BlockSpec((1, half), lambda i: (0, 0)),
            pl.BlockSpec((F, H), lambda i: (0, 0)),
            pl.BlockSpec((1, H), lambda i: (0, 0)),
            pl.BlockSpec((H, H), lambda i: (0, 0)),
            pl.BlockSpec((1, H), lambda i: (0, 0)),
        ],
        out_specs=pl.BlockSpec((tn, H), lambda i: (i, 0)),
        out_shape=jax.ShapeDtypeStruct((n_pad, H), jnp.float32),
        compiler_params=pltpu.CompilerParams(
            dimension_semantics=("parallel",)),
    )(t_col, freqs, w1_bf, b1.reshape(1, H), w2_bf, b2.reshape(1, H))
    return out[:N]


# speedup vs baseline: 1.0937x; 1.0937x over previous
import math

import jax
import jax.numpy as jnp
from jax.experimental import pallas as pl
from jax.experimental.pallas import tpu as pltpu


def _embedder_kernel(t_ref, freqs_ref, w1f_ref, b1_ref, w2f_ref, b2_ref,
                     o_ref, w1b_ref, w2b_ref, *, sub_rows):
    @pl.when(pl.program_id(1) == 0)
    def _cast_weights_once():
        w1b_ref[...] = w1f_ref[...].astype(jnp.bfloat16)
        w2b_ref[...] = w2f_ref[...].astype(jnp.bfloat16)

    tn = t_ref.shape[0]
    half = freqs_ref.shape[1]
    freqs = freqs_ref[...]
    b1 = b1_ref[...]
    b2 = b2_ref[...]
    w1c = w1b_ref[:half, :]
    w1s = w1b_ref[half:, :]
    w2 = w2b_ref[...]

    for c in range(tn // sub_rows):
        sl = pl.ds(c * sub_rows, sub_rows)
        args = t_ref[sl, :] * freqs
        cos_b = jnp.cos(args).astype(jnp.bfloat16)
        sin_b = jnp.sin(args).astype(jnp.bfloat16)
        h = (jnp.dot(cos_b, w1c, preferred_element_type=jnp.float32)
             + jnp.dot(sin_b, w1s, preferred_element_type=jnp.float32)
             + b1)
        h = (h * jax.lax.logistic(h)).astype(jnp.bfloat16)
        o_ref[sl, :] = (jnp.dot(h, w2, preferred_element_type=jnp.float32)
                        + b2)


def kernel(t, w1, b1, w2, b2, *, frequency_embedding_size=256,
           max_period=10000, max_tile_n=512, sub_rows=128):
    N = t.shape[0]
    F = frequency_embedding_size
    half = F // 2
    H = w1.shape[1]
    assert F % 2 == 0, "frequency_embedding_size must be even"
    assert w1.shape[0] == F and w2.shape == (H, H)

    freqs = jnp.exp(
        -math.log(max_period) * jnp.arange(half, dtype=jnp.float32) / half
    ).reshape(1, half)

    tn = min(max_tile_n, -(-N // 8) * 8)
    sub = sub_rows if tn % sub_rows == 0 else tn
    n_pad = -(-N // tn) * tn
    if n_pad == N:
        t_col = t.astype(jnp.float32).reshape(N, 1)
    else:
        t_col = jnp.zeros((n_pad, 1), jnp.float32).at[:N, 0].set(
            t.astype(jnp.float32))

    n_tiles = n_pad // tn
    outer = 2 if n_tiles % 2 == 0 else 1
    inner = n_tiles // outer

    from functools import partial
    out = pl.pallas_call(
        partial(_embedder_kernel, sub_rows=sub),
        grid=(outer, inner),
        in_specs=[
            pl.BlockSpec((tn, 1), lambda i, j: (i * inner + j, 0)),
            pl.BlockSpec((1, half), lambda i, j: (0, 0)),
            pl.BlockSpec((F, H), lambda i, j: (0, 0)),
            pl.BlockSpec((1, H), lambda i, j: (0, 0)),
            pl.BlockSpec((H, H), lambda i, j: (0, 0)),
            pl.BlockSpec((1, H), lambda i, j: (0, 0)),
        ],
        out_specs=pl.BlockSpec((tn, H), lambda i, j: (i * inner + j, 0)),
        out_shape=jax.ShapeDtypeStruct((n_pad, H), jnp.float32),
        scratch_shapes=[
            pltpu.VMEM((F, H), jnp.bfloat16),
            pltpu.VMEM((H, H), jnp.bfloat16),
        ],
        compiler_params=pltpu.CompilerParams(
            dimension_semantics=("parallel", "arbitrary")),
    )(t_col, freqs, w1, b1.reshape(1, H), w2, b2.reshape(1, H))
    return out[:N]


# speedup vs baseline: 1.1548x; 1.0558x over previous
import math
from functools import partial

import jax
import jax.numpy as jnp
from jax.experimental import pallas as pl
from jax.experimental.pallas import tpu as pltpu


def _embedder_kernel(t_ref, freqs_ref, w1_ref, b1_ref, w2_ref,
                     b2_ref, o_ref, *, sub_rows):
    tn = t_ref.shape[0]
    half = freqs_ref.shape[1]
    freqs = freqs_ref[...]
    b1 = b1_ref[...]
    b2 = b2_ref[...]
    w1c = w1_ref[:half, :]
    w1s = w1_ref[half:, :]
    w2 = w2_ref[...]

    for c in range(tn // sub_rows):
        sl = pl.ds(c * sub_rows, sub_rows)
        args = t_ref[sl, :] * freqs
        h = (jnp.dot(jnp.cos(args), w1c, preferred_element_type=jnp.float32)
             + jnp.dot(jnp.sin(args), w1s, preferred_element_type=jnp.float32)
             + b1)
        h = h * jax.lax.logistic(h)
        o_ref[sl, :] = (jnp.dot(h, w2, preferred_element_type=jnp.float32)
                        + b2)


def kernel(t, w1, b1, w2, b2, *, frequency_embedding_size=256,
           max_period=10000, max_tile_n=512, sub_rows=128):
    N = t.shape[0]
    F = frequency_embedding_size
    half = F // 2
    H = w1.shape[1]
    assert F % 2 == 0, "frequency_embedding_size must be even"
    assert w1.shape[0] == F and w2.shape == (H, H)

    freqs = jnp.exp(
        -math.log(max_period) * jnp.arange(half, dtype=jnp.float32) / half
    ).reshape(1, half)

    tn = min(max_tile_n, -(-N // 8) * 8)
    sub = sub_rows if tn % sub_rows == 0 else tn
    n_pad = -(-N // tn) * tn
    if n_pad == N:
        t_col = t.astype(jnp.float32).reshape(N, 1)
    else:
        t_col = jnp.zeros((n_pad, 1), jnp.float32).at[:N, 0].set(
            t.astype(jnp.float32))

    out = pl.pallas_call(
        partial(_embedder_kernel, sub_rows=sub),
        grid=(n_pad // tn,),
        in_specs=[
            pl.BlockSpec((tn, 1), lambda i: (i, 0)),
            pl.BlockSpec((1, half), lambda i: (0, 0)),
            pl.BlockSpec((F, H), lambda i: (0, 0)),
            pl.BlockSpec((1, H), lambda i: (0, 0)),
            pl.BlockSpec((H, H), lambda i: (0, 0)),
            pl.BlockSpec((1, H), lambda i: (0, 0)),
        ],
        out_specs=pl.BlockSpec((tn, H), lambda i: (i, 0)),
        out_shape=jax.ShapeDtypeStruct((n_pad, H), jnp.float32),
        compiler_params=pltpu.CompilerParams(
            dimension_semantics=("arbitrary",)),
    )(t_col, freqs, w1, b1.reshape(1, H), w2, b2.reshape(1, H))
    return out[:N]
